# baseline (speedup 1.0000x reference)
import functools

import jax
import jax.numpy as jnp
from jax import lax
from jax.experimental import pallas as pl
from jax.experimental.pallas import tpu as pltpu
from jax.experimental.pallas import tpu_sc as plsc

B = 16384
EMB = 32
NC = 2
NS = 16
NW = NC * NS
B_PER_W = B // NW
CHUNK = 128
N_CHUNKS = B_PER_W // CHUNK

_sc_mesh = plsc.VectorSubcoreMesh(core_axis_name="c", subcore_axis_name="s")


def _repack_body(tT, out):
    out[...] = jnp.swapaxes(tT[...], 0, 1)


def _repack(tT, rows):
    blk = 2048
    grid = (rows + blk - 1) // blk
    return pl.pallas_call(
        _repack_body,
        grid=(grid,),
        in_specs=[pl.BlockSpec((EMB, blk), lambda n: (0, n))],
        out_specs=pl.BlockSpec((blk, EMB), lambda n: (n, 0)),
        out_shape=jax.ShapeDtypeStruct((rows, EMB), jnp.float32),
    )(tT)


@functools.partial(
    pl.kernel,
    out_type=[
        jax.ShapeDtypeStruct((B, EMB), jnp.float32),
        jax.ShapeDtypeStruct((B, EMB), jnp.float32),
    ],
    mesh=_sc_mesh,
    scratch_types=[
        pltpu.VMEM((N_CHUNKS, CHUNK), jnp.int32),
        pltpu.VMEM((N_CHUNKS, CHUNK), jnp.int32),
        pltpu.VMEM((B_PER_W, EMB), jnp.float32),
        pltpu.VMEM((B_PER_W, EMB), jnp.float32),
        pltpu.SemaphoreType.DMA,
        pltpu.SemaphoreType.DMA,
    ],
    compiler_params=pltpu.CompilerParams(use_tc_tiling_on_sc=False),
)
def _sc_gather(u_idx_hbm, i_idx_hbm, user_hbm, item_hbm,
               u_out, i_out,
               u_idx_v, i_idx_v, u_rows, i_rows, sem_u, sem_i):
    wid = lax.axis_index("s") * NC + lax.axis_index("c")
    base = wid * B_PER_W
    pltpu.sync_copy(u_idx_hbm.at[wid], u_idx_v)
    pltpu.sync_copy(i_idx_hbm.at[wid], i_idx_v)
    copies = []
    for j in range(N_CHUNKS):
        copies.append(pltpu.async_copy(
            user_hbm.at[u_idx_v.at[j]],
            u_rows.at[pl.ds(j * CHUNK, CHUNK)], sem_u))
        copies.append(pltpu.async_copy(
            item_hbm.at[i_idx_v.at[j]],
            i_rows.at[pl.ds(j * CHUNK, CHUNK)], sem_i))
    for c in copies:
        c.wait()
    pltpu.sync_copy(u_rows, u_out.at[pl.ds(base, B_PER_W)])
    pltpu.sync_copy(i_rows, i_out.at[pl.ds(base, B_PER_W)])


def _mlp_body(u_emb, i_emb, W1u, W1i, b1, W2, b2, W3, b3, W4, b4, out):
    h = jnp.maximum(
        u_emb[...] @ W1u[...] + i_emb[...] @ W1i[...] + b1[...], 0.0)
    h = jnp.maximum(h @ W2[...] + b2[...], 0.0)
    h = jnp.maximum(h @ W3[...] + b3[...], 0.0)
    out[...] = jax.nn.sigmoid(h @ W4[...] + b4[...])


def _mlp(u_emb, i_emb, W1u, W1i, b1, W2, b2, W3, b3, W4, b4):
    rows = 2048
    grid = B // rows
    full = lambda shape: pl.BlockSpec(shape, lambda n: (0, 0))
    return pl.pallas_call(
        _mlp_body,
        grid=(grid,),
        in_specs=[
            pl.BlockSpec((rows, EMB), lambda n: (n, 0)),
            pl.BlockSpec((rows, EMB), lambda n: (n, 0)),
            full((EMB, 32)), full((EMB, 32)), full((1, 32)),
            full((32, 16)), full((1, 16)),
            full((16, 8)), full((1, 8)),
            full((8, 1)), full((1, 1)),
        ],
        out_specs=pl.BlockSpec((rows, 1), lambda n: (n, 0)),
        out_shape=jax.ShapeDtypeStruct((B, 1), jnp.float32),
    )(u_emb, i_emb, W1u, W1i, b1, W2, b2, W3, b3, W4, b4)


@jax.jit
def kernel(u, i, user_table, item_table, W1, b1, W2, b2, W3, b3, W4, b4):
    u_idx = u.astype(jnp.int32).reshape(NW, N_CHUNKS, CHUNK)
    i_idx = i.astype(jnp.int32).reshape(NW, N_CHUNKS, CHUNK)
    user_lin = _repack(user_table.T, user_table.shape[0])
    item_lin = _repack(item_table.T, item_table.shape[0])
    u_emb, i_emb = _sc_gather(u_idx, i_idx, user_lin, item_lin)
    out = _mlp(
        u_emb, i_emb,
        W1[:EMB], W1[EMB:], b1.reshape(1, 32),
        W2, b2.reshape(1, 16),
        W3, b3.reshape(1, 8),
        W4, b4.reshape(1, 1),
    )
    return out.reshape(B)


# speedup vs baseline: 2.3467x; 2.3467x over previous
import functools

import jax
import jax.numpy as jnp
from jax import lax
from jax.experimental import pallas as pl
from jax.experimental.pallas import tpu as pltpu
from jax.experimental.pallas import tpu_sc as plsc

B = 16384
EMB = 32
PACK = 4
PW = EMB * PACK
NC = 2
NS = 16
NW = NC * NS
B_PER_W = B // NW
CHUNK = 128
N_CHUNKS = B_PER_W // CHUNK

_sc_mesh = plsc.VectorSubcoreMesh(core_axis_name="c", subcore_axis_name="s")


SB = 2048
CB = PACK * SB


def _repack_body(tT, out):
    x = tT[...]
    out[...] = jnp.concatenate(
        [jnp.swapaxes(x[:, m * SB:(m + 1) * SB], 0, 1) for m in range(PACK)],
        axis=1)


def _repack(tT, rows):
    grid = (rows + CB - 1) // CB
    return pl.pallas_call(
        _repack_body,
        grid=(grid,),
        in_specs=[pl.BlockSpec((EMB, CB), lambda n: (0, n))],
        out_specs=pl.BlockSpec((SB, PW), lambda n: (n, 0)),
        out_shape=jax.ShapeDtypeStruct((grid * SB, PW), jnp.float32),
    )(tT)


@functools.partial(
    pl.kernel,
    out_type=[
        jax.ShapeDtypeStruct((B, PW), jnp.float32),
        jax.ShapeDtypeStruct((B, PW), jnp.float32),
    ],
    mesh=_sc_mesh,
    scratch_types=[
        pltpu.VMEM((N_CHUNKS, CHUNK), jnp.int32),
        pltpu.VMEM((B_PER_W, PW), jnp.float32),
        pltpu.SemaphoreType.DMA,
    ],
    compiler_params=pltpu.CompilerParams(use_tc_tiling_on_sc=False),
)
def _sc_gather(u_idx_hbm, i_idx_hbm, user_hbm, item_hbm,
               u_out, i_out,
               idx_v, rows_v, sem):
    wid = lax.axis_index("s") * NC + lax.axis_index("c")
    base = wid * B_PER_W
    for idx_hbm, tab_hbm, out in (
            (u_idx_hbm, user_hbm, u_out), (i_idx_hbm, item_hbm, i_out)):
        pltpu.sync_copy(idx_hbm.at[wid], idx_v)
        copies = []
        for j in range(N_CHUNKS):
            copies.append(pltpu.async_copy(
                tab_hbm.at[idx_v.at[j]],
                rows_v.at[pl.ds(j * CHUNK, CHUNK)], sem))
        for c in copies:
            c.wait()
        pltpu.sync_copy(rows_v, out.at[pl.ds(base, B_PER_W)])


def _mlp_body(u_big, i_big, uu, ii, W1u, W1i, b1, W2, b2, W3, b3, W4, b4,
              out):
    lane = jax.lax.broadcasted_iota(jnp.int32, u_big.shape, 1) // EMB
    u_sel = jnp.where(lane == uu[...], u_big[...], 0.0)
    i_sel = jnp.where(lane == ii[...], i_big[...], 0.0)
    h = jnp.maximum(
        u_sel @ W1u[...] + i_sel @ W1i[...] + b1[...], 0.0)
    h = jnp.maximum(h @ W2[...] + b2[...], 0.0)
    h = jnp.maximum(h @ W3[...] + b3[...], 0.0)
    out[...] = jax.nn.sigmoid(h @ W4[...] + b4[...])


def _mlp(u_big, i_big, uu, ii, W1u, W1i, b1, W2, b2, W3, b3, W4, b4):
    rows = 2048
    grid = B // rows
    full = lambda shape: pl.BlockSpec(shape, lambda n: (0, 0))
    return pl.pallas_call(
        _mlp_body,
        grid=(grid,),
        in_specs=[
            pl.BlockSpec((rows, PW), lambda n: (n, 0)),
            pl.BlockSpec((rows, PW), lambda n: (n, 0)),
            pl.BlockSpec((rows, 1), lambda n: (n, 0)),
            pl.BlockSpec((rows, 1), lambda n: (n, 0)),
            full((PW, 32)), full((PW, 32)), full((1, 32)),
            full((32, 16)), full((1, 16)),
            full((16, 8)), full((1, 8)),
            full((8, 1)), full((1, 1)),
        ],
        out_specs=pl.BlockSpec((rows, 1), lambda n: (n, 0)),
        out_shape=jax.ShapeDtypeStruct((B, 1), jnp.float32),
    )(u_big, i_big, uu, ii, W1u, W1i, b1, W2, b2, W3, b3, W4, b4)


@jax.jit
def kernel(u, i, user_table, item_table, W1, b1, W2, b2, W3, b3, W4, b4):
    u32 = u.astype(jnp.int32)
    i32 = i.astype(jnp.int32)
    u_idx = ((u32 // CB) * SB + u32 % SB).reshape(NW, N_CHUNKS, CHUNK)
    i_idx = ((i32 // CB) * SB + i32 % SB).reshape(NW, N_CHUNKS, CHUNK)
    user_pk = _repack(user_table.T, user_table.shape[0])
    item_pk = _repack(item_table.T, item_table.shape[0])
    u_big, i_big = _sc_gather(u_idx, i_idx, user_pk, item_pk)
    out = _mlp(
        u_big, i_big, (u32 // SB) % PACK, (i32 // SB) % PACK,
        jnp.tile(W1[:EMB], (PACK, 1)), jnp.tile(W1[EMB:], (PACK, 1)),
        b1.reshape(1, 32),
        W2, b2.reshape(1, 16),
        W3, b3.reshape(1, 8),
        W4, b4.reshape(1, 1),
    )
    return out.reshape(B)


# speedup vs baseline: 2.8863x; 1.2299x over previous
import functools

import jax
import jax.numpy as jnp
from jax import lax
from jax.experimental import pallas as pl
from jax.experimental.pallas import tpu as pltpu
from jax.experimental.pallas import tpu_sc as plsc

B = 16384
EMB = 32
PACK = 4
PW = EMB * PACK
NC = 2
NS = 16
NW = NC * NS
B_PER_W = B // NW
CHUNK = 128
N_CHUNKS = B_PER_W // CHUNK
SB = 2048
CB = PACK * SB

_sc_mesh = plsc.VectorSubcoreMesh(core_axis_name="c", subcore_axis_name="s")


def _repack_body(tT, eye, out):
    x = tT[...]
    e = eye[...]
    dn = (((0,), (0,)), ((), ()))
    out[...] = jnp.concatenate(
        [lax.dot_general(x[:, m * SB:(m + 1) * SB], e, dn,
                         preferred_element_type=jnp.float32)
         for m in range(PACK)],
        axis=1)


def _repack(tT, eye, rows):
    grid = (rows + CB - 1) // CB
    return pl.pallas_call(
        _repack_body,
        grid=(grid,),
        in_specs=[pl.BlockSpec((EMB, CB), lambda n: (0, n)),
                  pl.BlockSpec((EMB, EMB), lambda n: (0, 0))],
        out_specs=pl.BlockSpec((SB, PW), lambda n: (n, 0)),
        out_shape=jax.ShapeDtypeStruct((grid * SB, PW), jnp.float32),
    )(tT, eye)


@functools.partial(
    pl.kernel,
    out_type=[
        jax.ShapeDtypeStruct((B, EMB), jnp.float32),
        jax.ShapeDtypeStruct((B, EMB), jnp.float32),
    ],
    mesh=_sc_mesh,
    scratch_types=[
        pltpu.VMEM((N_CHUNKS, CHUNK), jnp.int32),
        pltpu.VMEM((B_PER_W, EMB), jnp.float32),
        pltpu.SemaphoreType.DMA,
    ],
    compiler_params=pltpu.CompilerParams(use_tc_tiling_on_sc=False),
)
def _sc_gather(u_idx_hbm, i_idx_hbm, user_hbm, item_hbm,
               u_out, i_out, idx_v, rows_v, sem):
    wid = lax.axis_index("s") * NC + lax.axis_index("c")
    base = wid * B_PER_W
    for idx_hbm, tab_hbm, out in (
            (u_idx_hbm, user_hbm, u_out), (i_idx_hbm, item_hbm, i_out)):
        pltpu.sync_copy(idx_hbm.at[wid], idx_v)
        copies = []
        for j in range(N_CHUNKS):
            copies.append(pltpu.async_copy(
                tab_hbm.at[idx_v.at[j]],
                rows_v.at[pl.ds(j * CHUNK, CHUNK)], sem))
        for c in copies:
            c.wait()
        pltpu.sync_copy(rows_v, out.at[pl.ds(base, B_PER_W)])


def _mlp_body(u4, i4, W1u, W1i, b1, W2, b2, W3, b3, W4, b4, out):
    h = jnp.maximum(u4[...] @ W1u[...] + i4[...] @ W1i[...] + b1[...], 0.0)
    h = jnp.maximum(h @ W2[...] + b2[...], 0.0)
    h = jnp.maximum(h @ W3[...] + b3[...], 0.0)
    out[...] = jax.nn.sigmoid(h @ W4[...] + b4[...])


def _mlp(u4, i4, W1u, W1i, b1, W2, b2, W3, b3, W4, b4):
    rows = 2048
    grid = (B // PACK) // rows
    full = lambda shape: pl.BlockSpec(shape, lambda n: (0, 0))
    return pl.pallas_call(
        _mlp_body,
        grid=(grid,),
        in_specs=[
            pl.BlockSpec((rows, PW), lambda n: (n, 0)),
            pl.BlockSpec((rows, PW), lambda n: (n, 0)),
            full((PW, PW)), full((PW, PW)), full((1, PW)),
            full((PW, 16 * PACK)), full((1, 16 * PACK)),
            full((16 * PACK, 8 * PACK)), full((1, 8 * PACK)),
            full((8 * PACK, PACK)), full((1, PACK)),
        ],
        out_specs=pl.BlockSpec((rows, PACK), lambda n: (n, 0)),
        out_shape=jax.ShapeDtypeStruct((B // PACK, PACK), jnp.float32),
    )(u4, i4, W1u, W1i, b1, W2, b2, W3, b3, W4, b4)


def _packed_idx(v):
    r = v >> 13
    t = v & (SB - 1)
    m = (v >> 11) & (PACK - 1)
    return (r * SB + t) * PACK + m


@jax.jit
def kernel(u, i, user_table, item_table, W1, b1, W2, b2, W3, b3, W4, b4):
    u1 = u.astype(jnp.int32).reshape(B)
    i1 = i.astype(jnp.int32).reshape(B)
    u_idx = _packed_idx(u1).reshape(NW, N_CHUNKS, CHUNK)
    i_idx = _packed_idx(i1).reshape(NW, N_CHUNKS, CHUNK)
    eye = jnp.eye(EMB, dtype=jnp.float32)
    user_pk = _repack(user_table.T, eye, user_table.shape[0])
    item_pk = _repack(item_table.T, eye, item_table.shape[0])
    u_emb, i_emb = _sc_gather(
        u_idx, i_idx,
        user_pk.reshape(-1, EMB), item_pk.reshape(-1, EMB))
    I4 = jnp.eye(PACK, dtype=jnp.float32)
    kron = lambda W: jnp.kron(I4, W)
    tile = lambda b: jnp.tile(b, PACK).reshape(1, -1)
    out = _mlp(
        u_emb.reshape(B // PACK, PW), i_emb.reshape(B // PACK, PW),
        kron(W1[:EMB]), kron(W1[EMB:]), tile(b1),
        kron(W2), tile(b2),
        kron(W3), tile(b3),
        kron(W4), tile(b4),
    )
    return out.reshape(B)


# speedup vs baseline: 4.7886x; 1.6591x over previous
import functools

import jax
import jax.numpy as jnp
from jax import lax
from jax.experimental import pallas as pl
from jax.experimental.pallas import tpu as pltpu
from jax.experimental.pallas import tpu_sc as plsc

B = 16384
EMB = 32
PACK = 4
PW = EMB * PACK
NC = 2
NS = 16
NW = NC * NS
B_PER_W = B // NW
CHUNK = 128
N_CHUNKS = B_PER_W // CHUNK
SB = 2048
CB = PACK * SB

_sc_mesh = plsc.VectorSubcoreMesh(core_axis_name="c", subcore_axis_name="s")


def _repack_body(tT, eye, out):
    del eye
    x = tT[...]
    xs = jnp.concatenate(
        [x[:, m * SB:(m + 1) * SB] for m in range(PACK)], axis=0)
    out[...] = jnp.swapaxes(xs, 0, 1)


def _repack(tT, eye, rows):
    grid = (rows + CB - 1) // CB
    return pl.pallas_call(
        _repack_body,
        grid=(grid,),
        in_specs=[pl.BlockSpec((EMB, CB), lambda n: (0, n)),
                  pl.BlockSpec((EMB, EMB), lambda n: (0, 0))],
        out_specs=pl.BlockSpec((SB, PW), lambda n: (n, 0)),
        out_shape=jax.ShapeDtypeStruct((grid * SB, PW), jnp.float32),
    )(tT, eye)


@functools.partial(
    pl.kernel,
    out_type=[
        jax.ShapeDtypeStruct((B, EMB), jnp.float32),
        jax.ShapeDtypeStruct((B, EMB), jnp.float32),
    ],
    mesh=_sc_mesh,
    scratch_types=[
        pltpu.VMEM((N_CHUNKS, CHUNK), jnp.int32),
        pltpu.VMEM((B_PER_W, EMB), jnp.float32),
        pltpu.SemaphoreType.DMA,
    ],
    compiler_params=pltpu.CompilerParams(use_tc_tiling_on_sc=False),
)
def _sc_gather(u_idx_hbm, i_idx_hbm, user_hbm, item_hbm,
               u_out, i_out, idx_v, rows_v, sem):
    wid = lax.axis_index("s") * NC + lax.axis_index("c")
    base = wid * B_PER_W
    for idx_hbm, tab_hbm, out in (
            (u_idx_hbm, user_hbm, u_out), (i_idx_hbm, item_hbm, i_out)):
        pltpu.sync_copy(idx_hbm.at[wid], idx_v)
        copies = []
        for j in range(N_CHUNKS):
            copies.append(pltpu.async_copy(
                tab_hbm.at[idx_v.at[j]],
                rows_v.at[pl.ds(j * CHUNK, CHUNK)], sem))
        for c in copies:
            c.wait()
        pltpu.sync_copy(rows_v, out.at[pl.ds(base, B_PER_W)])


def _mlp_body(u4, i4, W1u, W1i, b1, W2, b2, W3, b3, W4, b4, out):
    h = jnp.maximum(u4[...] @ W1u[...] + i4[...] @ W1i[...] + b1[...], 0.0)
    h = jnp.maximum(h @ W2[...] + b2[...], 0.0)
    h = jnp.maximum(h @ W3[...] + b3[...], 0.0)
    out[...] = jax.nn.sigmoid(h @ W4[...] + b4[...])


def _mlp(u4, i4, W1u, W1i, b1, W2, b2, W3, b3, W4, b4):
    rows = 2048
    grid = (B // PACK) // rows
    full = lambda shape: pl.BlockSpec(shape, lambda n: (0, 0))
    return pl.pallas_call(
        _mlp_body,
        grid=(grid,),
        in_specs=[
            pl.BlockSpec((rows, PW), lambda n: (n, 0)),
            pl.BlockSpec((rows, PW), lambda n: (n, 0)),
            full((PW, PW)), full((PW, PW)), full((1, PW)),
            full((PW, 16 * PACK)), full((1, 16 * PACK)),
            full((16 * PACK, 8 * PACK)), full((1, 8 * PACK)),
            full((8 * PACK, PACK)), full((1, PACK)),
        ],
        out_specs=pl.BlockSpec((rows, PACK), lambda n: (n, 0)),
        out_shape=jax.ShapeDtypeStruct((B // PACK, PACK), jnp.float32),
    )(u4, i4, W1u, W1i, b1, W2, b2, W3, b3, W4, b4)


def _packed_idx(v):
    r = v >> 13
    t = v & (SB - 1)
    m = (v >> 11) & (PACK - 1)
    return (r * SB + t) * PACK + m


@jax.jit
def kernel(u, i, user_table, item_table, W1, b1, W2, b2, W3, b3, W4, b4):
    u1 = u.astype(jnp.int32).reshape(B)
    i1 = i.astype(jnp.int32).reshape(B)
    u_idx = _packed_idx(u1).reshape(NW, N_CHUNKS, CHUNK)
    i_idx = _packed_idx(i1).reshape(NW, N_CHUNKS, CHUNK)
    eye = jnp.eye(EMB, dtype=jnp.float32)
    user_pk = _repack(user_table.T, eye, user_table.shape[0])
    item_pk = _repack(item_table.T, eye, item_table.shape[0])
    u_emb, i_emb = _sc_gather(
        u_idx, i_idx,
        user_pk.reshape(-1, EMB), item_pk.reshape(-1, EMB))
    I4 = jnp.eye(PACK, dtype=jnp.float32)
    kron = lambda W: jnp.kron(I4, W)
    tile = lambda b: jnp.tile(b, PACK).reshape(1, -1)
    out = _mlp(
        u_emb.reshape(B // PACK, PW), i_emb.reshape(B // PACK, PW),
        kron(W1[:EMB]), kron(W1[EMB:]), tile(b1),
        kron(W2), tile(b2),
        kron(W3), tile(b3),
        kron(W4), tile(b4),
    )
    return out.reshape(B)
